# baseline (speedup 1.0000x reference)
import functools

import jax
import jax.numpy as jnp
from jax import lax
from jax.experimental import pallas as pl
from jax.experimental.pallas import tpu as pltpu
from jax.experimental.pallas import tpu_sc as plsc

ROWS, COLS = 16384, 128

_info = plsc.get_sparse_core_info()
_NC, _NS, _L = _info.num_cores, _info.num_subcores, _info.num_lanes
_NW = _NC * _NS
_RPW = ROWS // _NW

_mesh = plsc.VectorSubcoreMesh(core_axis_name="c", subcore_axis_name="s")


@functools.partial(
    pl.kernel,
    mesh=_mesh,
    out_type=jax.ShapeDtypeStruct((ROWS, COLS), jnp.float32),
    scratch_types=[pltpu.VMEM((COLS,), jnp.float32)],
)
def _copy_assign(x_hbm, out_hbm, row_v):
    wid = lax.axis_index("s") * _NC + lax.axis_index("c")
    base = wid * _RPW
    pltpu.sync_copy(x_hbm.at[pl.ds(base, _RPW)], out_hbm.at[pl.ds(base, _RPW)])

    @pl.when(wid == 0)
    def _patch():
        pltpu.sync_copy(x_hbm.at[1], row_v)
        v = row_v[pl.ds(0, _L)]
        v = jnp.where(lax.iota(jnp.int32, _L) == 2, jnp.float32(1.0), v)
        row_v[pl.ds(0, _L)] = v
        pltpu.sync_copy(row_v, out_hbm.at[1])


def kernel(x):
    return _copy_assign(x)


# speedup vs baseline: 10.9677x; 10.9677x over previous
import functools

import jax
import jax.numpy as jnp
from jax import lax
from jax.experimental import pallas as pl
from jax.experimental.pallas import tpu as pltpu
from jax.experimental.pallas import tpu_sc as plsc

ROWS, COLS = 16384, 128

_info = plsc.get_sparse_core_info()
_NC, _NS, _L = _info.num_cores, _info.num_subcores, _info.num_lanes
_NW = _NC * _NS
_RPW = ROWS // _NW

_mesh = plsc.VectorSubcoreMesh(core_axis_name="c", subcore_axis_name="s")


@functools.partial(
    pl.kernel,
    mesh=_mesh,
    out_type=jax.ShapeDtypeStruct((ROWS, COLS), jnp.float32),
    scratch_types=[pltpu.VMEM((_RPW, COLS), jnp.float32)],
)
def _copy_assign(x_hbm, out_hbm, buf):
    wid = lax.axis_index("s") * _NC + lax.axis_index("c")
    base = wid * _RPW
    pltpu.sync_copy(x_hbm.at[pl.ds(base, _RPW)], buf)

    @pl.when(wid == 0)
    def _patch():
        v = buf[1, pl.ds(0, _L)]
        buf[1, pl.ds(0, _L)] = jnp.where(
            lax.iota(jnp.int32, _L) == 2, jnp.float32(1.0), v)

    pltpu.sync_copy(buf, out_hbm.at[pl.ds(base, _RPW)])


def kernel(x):
    return _copy_assign(x)
